# Initial kernel scaffold; baseline (speedup 1.0000x reference)
import jax
import jax.numpy as jnp
from jax.experimental import pallas as pl


def kernel(q_len, k_len, table):
    raise NotImplementedError("write your pallas kernel here")


# speedup vs baseline: 136.4774x; 136.4774x over previous
import jax
import jax.numpy as jnp
from jax.experimental import pallas as pl
from jax.experimental.pallas import tpu as pltpu

_NUM_BUCKETS = 32
_N_HEADS = 16
_Q_LEN = 2048
_K_LEN = 2048
_BQ = 256
_W = 4096
_THRESHOLDS = (12, 16, 23, 32, 46, 64, 91)


def _bias_kernel(delta_ref, table_ref, out_ref, f8_ref, f_ref):
    i = pl.program_id(1)

    @pl.when(i == 0)
    def _build():
        delta = delta_ref[0]
        lane = jax.lax.broadcasted_iota(jnp.int32, (8, _W + 128), 1)
        sub = jax.lax.broadcasted_iota(jnp.int32, (8, _W + 128), 0)
        rel = lane - sub - 128 - (_Q_LEN - 128) + delta
        ret = jnp.where(rel > 0, _NUM_BUCKETS // 2, 0)
        rp = jnp.abs(rel)
        large = jnp.full(rel.shape, 8, jnp.int32)
        for t in _THRESHOLDS:
            large = large + (rp >= t).astype(jnp.int32)
        bucket = ret + jnp.where(rp < 8, rp, large)
        acc = jnp.zeros((8, _W + 128), jnp.float32)
        for b in range(_NUM_BUCKETS):
            acc = acc + (bucket == b).astype(jnp.float32) * table_ref[0, 0, b]
        f8_ref[:, :] = acc
        for k in range(16):
            f_ref[8 * k:8 * (k + 1), :] = f8_ref[:, 128 - 8 * k:128 - 8 * k + _W]

    for u in range(_BQ // 128):
        m = 15 - 2 * i - u
        out_ref[0, 0, 128 * u:128 * (u + 1), :] = f_ref[
            :, pl.ds(pl.multiple_of(128 * m, 128), _K_LEN)
        ]


def kernel(q_len, k_len, table):
    delta = (jnp.asarray(k_len, jnp.int32) - _K_LEN) - (
        jnp.asarray(q_len, jnp.int32) - _Q_LEN
    )
    table_t = jnp.reshape(jnp.transpose(table), (_N_HEADS, 1, _NUM_BUCKETS))
    grid_spec = pltpu.PrefetchScalarGridSpec(
        num_scalar_prefetch=1,
        grid=(_N_HEADS, _Q_LEN // _BQ),
        in_specs=[
            pl.BlockSpec((1, 1, _NUM_BUCKETS), lambda h, i, *_: (h, 0, 0)),
        ],
        out_specs=pl.BlockSpec(
            (1, 1, _BQ, _K_LEN), lambda h, i, *_: (0, h, i, 0)
        ),
        scratch_shapes=[
            pltpu.VMEM((8, _W + 128), jnp.float32),
            pltpu.VMEM((128, _W), jnp.float32),
        ],
    )
    out = pl.pallas_call(
        _bias_kernel,
        grid_spec=grid_spec,
        out_shape=jax.ShapeDtypeStruct(
            (1, _N_HEADS, _Q_LEN, _K_LEN), jnp.float32
        ),
    )(jnp.reshape(delta, (1,)), table_t)
    return out
